# baseline (speedup 1.0000x reference)
import functools

import jax
import jax.numpy as jnp
from jax import lax
from jax.experimental import pallas as pl
from jax.experimental.pallas import tpu as pltpu
from jax.experimental.pallas import tpu_sc as plsc

N_NODES = 10000
N_PAD = 10240
N_EDGES = 320000
D = 128

NC, NS, L = 2, 16, 16
NW = NC * NS
EPW = N_EDGES // NW
C = 80
NCHUNK = EPW // C
NBUF = 2
NCP = 128
ROWS_PT = N_PAD // NS


def _sc_aggregate(x, src, dst, ew):
    mesh = plsc.VectorSubcoreMesh(core_axis_name="c", subcore_axis_name="s")

    @functools.partial(
        pl.kernel,
        out_type=jax.ShapeDtypeStruct((NC, N_PAD, D), jnp.float32),
        mesh=mesh,
        scratch_types=[
            pltpu.VMEM((EPW,), jnp.int32),
            pltpu.VMEM((EPW,), jnp.float32),
            pltpu.VMEM((2, C), jnp.int32),
            pltpu.VMEM((2, C, D), jnp.float32),
            pltpu.VMEM_SHARED((N_PAD, D), jnp.float32),
            [pltpu.SemaphoreType.DMA] * 2,
            [pltpu.SemaphoreType.DMA] * 2,
            [pltpu.SemaphoreType.DMA] * 2,
            [pltpu.SemaphoreType.DMA] * 2,
            pltpu.SemaphoreType.DMA,
        ],
    )
    def agg(x_hbm, src_hbm, dst_hbm, ew_hbm, out_hbm,
            idx_s, ew_s, dst_v, rows_v, acc,
            semG, semS, semD, semT, semZ):
        cid = lax.axis_index("c")
        sid = lax.axis_index("s")
        wid = cid * NS + sid

        zero16 = jnp.zeros((L,), jnp.float32)

        def zstore(i, _):
            for k in range(D // L):
                rows_v[0, i, pl.ds(k * L, L)] = zero16
            return 0

        lax.fori_loop(0, C, zstore, 0)
        rbase = sid * ROWS_PT
        for j in range(ROWS_PT // C):
            pltpu.async_copy(rows_v.at[0], acc.at[pl.ds(rbase + j * C, C)],
                             semZ)

        sb = pl.multiple_of(wid * EPW, 8)
        pltpu.async_copy(src_hbm.at[pl.ds(sb, EPW)], idx_s, semT[0])
        pltpu.async_copy(ew_hbm.at[pl.ds(sb, EPW)], ew_s, semT[1])

        def load_dst(g, buf):
            b = pl.multiple_of(wid * EPW + g * C, 8)
            pltpu.async_copy(dst_hbm.at[pl.ds(b, C)], dst_v.at[buf], semD[buf])

        def wait_dst(buf):
            pltpu.make_async_copy(
                dst_hbm.at[pl.ds(0, C)], dst_v.at[buf], semD[buf]).wait()

        def start_gather(g, buf):
            gb = pl.multiple_of(g * C, 8)
            pltpu.async_copy(x_hbm.at[idx_s.at[pl.ds(gb, C)]], rows_v.at[buf],
                             semG[buf])

        def wait_gather(buf):
            pltpu.make_async_copy(
                x_hbm.at[idx_s.at[pl.ds(0, C)]], rows_v.at[buf],
                semG[buf]).wait()

        def start_scatter(buf):
            pltpu.async_copy(rows_v.at[buf], acc.at[dst_v.at[buf]], semS[buf],
                             add=True)

        def wait_scatter(buf):
            pltpu.make_async_copy(
                rows_v.at[buf], acc.at[dst_v.at[buf]], semS[buf]).wait()

        def scale(g, buf):
            def sbody(gg, _):
                wvec = ew_s[pl.ds(g * C + gg * L, L)]
                for j in range(L):
                    w = wvec[j]
                    e = gg * L + j
                    for k in range(D // L):
                        sl = pl.ds(k * L, L)
                        rows_v[buf, e, sl] = rows_v[buf, e, sl] * w
                return 0

            lax.fori_loop(0, C // L, sbody, 0)

        def steady(g, bb, first=False, last=False):
            nb = 1 - bb
            if not last:
                if not first:
                    wait_scatter(nb)
                load_dst(g + 1, nb)
                start_gather(g + 1, nb)
            wait_gather(bb)
            scale(g, bb)
            wait_dst(bb)
            start_scatter(bb)

        load_dst(0, 0)
        pltpu.make_async_copy(src_hbm.at[pl.ds(0, EPW)], idx_s,
                              semT[0]).wait()
        pltpu.make_async_copy(ew_hbm.at[pl.ds(0, EPW)], ew_s,
                              semT[1]).wait()
        for j in range(ROWS_PT // C):
            pltpu.make_async_copy(rows_v.at[0],
                                  acc.at[pl.ds(rbase + j * C, C)], semZ).wait()
        plsc.subcore_barrier()
        start_gather(0, 0)

        steady(0, 0, first=True)

        @pl.loop(1, NCHUNK - 2, step=2)
        def _(g0):
            for j in range(2):
                steady(g0 + j, (1 + j) % 2)

        steady(NCHUNK - 2, (NCHUNK - 2) % 2)
        steady(NCHUNK - 1, (NCHUNK - 1) % 2, last=True)

        wait_scatter(0)
        wait_scatter(1)
        plsc.subcore_barrier()

        pltpu.sync_copy(acc.at[pl.ds(rbase, ROWS_PT)],
                        out_hbm.at[cid, pl.ds(rbase, ROWS_PT)])

    return agg(x, src, dst, ew)


def _tc_combine(parts, W, b):
    R = 1000

    def body(p_ref, w_ref, b_ref, o_ref):
        s = p_ref[0] + p_ref[1]
        o_ref[...] = (
            jnp.dot(s, w_ref[...], preferred_element_type=jnp.float32)
            + b_ref[...]
        )

    return pl.pallas_call(
        body,
        grid=(N_NODES // R,),
        in_specs=[
            pl.BlockSpec((NC, R, D), lambda i: (0, i, 0)),
            pl.BlockSpec((D, D), lambda i: (0, 0)),
            pl.BlockSpec((1, D), lambda i: (0, 0)),
        ],
        out_specs=pl.BlockSpec((R, D), lambda i: (i, 0)),
        out_shape=jax.ShapeDtypeStruct((N_NODES, D), jnp.float32),
    )(parts, W, b.reshape(1, D))


def kernel(x, edge_index, edge_weight, W, b):
    src = edge_index[0].astype(jnp.int32)
    dst = edge_index[1].astype(jnp.int32)
    parts = _sc_aggregate(x, src, dst, edge_weight)
    return _tc_combine(parts, W, b)


# speedup vs baseline: 1.1534x; 1.1534x over previous
import functools

import jax
import jax.numpy as jnp
from jax import lax
from jax.experimental import pallas as pl
from jax.experimental.pallas import tpu as pltpu
from jax.experimental.pallas import tpu_sc as plsc

N_NODES = 10000
N_PAD = 10240
N_EDGES = 320000
D = 128

NC, NS, L = 2, 16, 16
NW = NC * NS
EPW = N_EDGES // NW
C = 80
NCHUNK = EPW // C
NBUF = 4
ROWS_PT = N_PAD // NS


def _sc_aggregate(x, src, dst, ew):
    mesh = plsc.VectorSubcoreMesh(core_axis_name="c", subcore_axis_name="s")

    @functools.partial(
        pl.kernel,
        out_type=jax.ShapeDtypeStruct((NC, N_PAD, D), jnp.float32),
        mesh=mesh,
        scratch_types=[
            pltpu.VMEM((NBUF, C), jnp.int32),
            pltpu.VMEM((NBUF, C), jnp.float32),
            pltpu.VMEM((NBUF, C), jnp.int32),
            pltpu.VMEM((NBUF, C, D), jnp.float32),
            pltpu.VMEM_SHARED((N_PAD, D), jnp.float32),
            [pltpu.SemaphoreType.DMA] * NBUF,
            [pltpu.SemaphoreType.DMA] * NBUF,
            [pltpu.SemaphoreType.DMA] * NBUF,
            [pltpu.SemaphoreType.DMA] * NBUF,
            [pltpu.SemaphoreType.DMA] * NBUF,
        ],
    )
    def agg(x_hbm, src_hbm, dst_hbm, ew_hbm, out_hbm,
            idx_v, ew_v, dst_v, rows_v, acc,
            semG, semS, semI, semE, semD):
        cid = lax.axis_index("c")
        sid = lax.axis_index("s")
        wid = cid * NS + sid

        zero16 = jnp.zeros((L,), jnp.float32)

        def zstore(i, _):
            for k in range(D // L):
                rows_v[0, i, pl.ds(k * L, L)] = zero16
            return 0

        lax.fori_loop(0, C, zstore, 0)
        rbase = sid * ROWS_PT
        for j in range(ROWS_PT // C):
            pltpu.sync_copy(rows_v.at[0], acc.at[pl.ds(rbase + j * C, C)])
        plsc.subcore_barrier()

        def load_meta(g, buf):
            b = pl.multiple_of(wid * EPW + g * C, 8)
            pltpu.async_copy(src_hbm.at[pl.ds(b, C)], idx_v.at[buf], semI[buf])
            pltpu.async_copy(ew_hbm.at[pl.ds(b, C)], ew_v.at[buf], semE[buf])

        def load_dst(g, buf):
            b = pl.multiple_of(wid * EPW + g * C, 8)
            pltpu.async_copy(dst_hbm.at[pl.ds(b, C)], dst_v.at[buf], semD[buf])

        def wait_src(buf):
            pltpu.make_async_copy(
                src_hbm.at[pl.ds(0, C)], idx_v.at[buf], semI[buf]).wait()

        def wait_ew(buf):
            pltpu.make_async_copy(
                ew_hbm.at[pl.ds(0, C)], ew_v.at[buf], semE[buf]).wait()

        def wait_dst(buf):
            pltpu.make_async_copy(
                dst_hbm.at[pl.ds(0, C)], dst_v.at[buf], semD[buf]).wait()

        def start_gather(buf):
            pltpu.async_copy(x_hbm.at[idx_v.at[buf]], rows_v.at[buf],
                             semG[buf])

        def wait_gather(buf):
            pltpu.make_async_copy(
                x_hbm.at[idx_v.at[buf]], rows_v.at[buf], semG[buf]).wait()

        def start_scatter(buf):
            pltpu.async_copy(rows_v.at[buf], acc.at[dst_v.at[buf]], semS[buf],
                             add=True)

        def wait_scatter(buf):
            pltpu.make_async_copy(
                rows_v.at[buf], acc.at[dst_v.at[buf]], semS[buf]).wait()

        def scale(buf):
            def sbody(gg, _):
                wvec = ew_v[buf, pl.ds(gg * L, L)]
                for j in range(L):
                    w = wvec[j]
                    e = gg * L + j
                    for k in range(D // L):
                        sl = pl.ds(k * L, L)
                        rows_v[buf, e, sl] = rows_v[buf, e, sl] * w
                return 0

            lax.fori_loop(0, C // L, sbody, 0)

        load_meta(0, 0)
        load_meta(1, 1)
        load_meta(2, 2)
        load_dst(0, 0)
        wait_src(0)
        start_gather(0)
        wait_src(1)
        start_gather(1)

        @pl.loop(0, NCHUNK + 3, step=NBUF)
        def _(g0):
            for bb in range(NBUF):
                g = g0 + bb
                b1 = (bb + 1) % NBUF
                b2 = (bb + 2) % NBUF
                b3 = (bb + 3) % NBUF

                @pl.when(g < NCHUNK)
                def _():
                    @pl.when(g + 3 < NCHUNK)
                    def _():
                        load_meta(g + 3, b3)

                    @pl.when(g + 2 < NCHUNK)
                    def _():
                        @pl.when(g >= 2)
                        def _():
                            wait_scatter(b2)

                        wait_src(b2)
                        start_gather(b2)

                    @pl.when(g + 1 < NCHUNK)
                    def _():
                        load_dst(g + 1, b1)

                    wait_gather(bb)
                    wait_ew(bb)
                    scale(bb)
                    wait_dst(bb)
                    start_scatter(bb)

        for buf in range(NBUF):
            wait_scatter(buf)
        plsc.subcore_barrier()

        pltpu.sync_copy(acc.at[pl.ds(rbase, ROWS_PT)],
                        out_hbm.at[cid, pl.ds(rbase, ROWS_PT)])

    return agg(x, src, dst, ew)


def _tc_combine(parts, W, b):
    R = 1000

    def body(p_ref, w_ref, b_ref, o_ref):
        s = p_ref[0] + p_ref[1]
        o_ref[...] = (
            jnp.dot(s, w_ref[...], preferred_element_type=jnp.float32)
            + b_ref[...]
        )

    return pl.pallas_call(
        body,
        grid=(N_NODES // R,),
        in_specs=[
            pl.BlockSpec((NC, R, D), lambda i: (0, i, 0)),
            pl.BlockSpec((D, D), lambda i: (0, 0)),
            pl.BlockSpec((1, D), lambda i: (0, 0)),
        ],
        out_specs=pl.BlockSpec((R, D), lambda i: (i, 0)),
        out_shape=jax.ShapeDtypeStruct((N_NODES, D), jnp.float32),
    )(parts, W, b.reshape(1, D))


def kernel(x, edge_index, edge_weight, W, b):
    src = edge_index[0].astype(jnp.int32)
    dst = edge_index[1].astype(jnp.int32)
    parts = _sc_aggregate(x, src, dst, edge_weight)
    return _tc_combine(parts, W, b)


# speedup vs baseline: 1.2366x; 1.0722x over previous
import functools

import jax
import jax.numpy as jnp
from jax import lax
from jax.experimental import pallas as pl
from jax.experimental.pallas import tpu as pltpu
from jax.experimental.pallas import tpu_sc as plsc

N_NODES = 10000
N_PAD = 10240
N_EDGES = 320000
D = 128

NC, NS, L = 2, 16, 16
NW = NC * NS
EPW = N_EDGES // NW
C = 80
NCHUNK = EPW // C
NBUF = 4
ROWS_PT = N_PAD // NS


def _sc_aggregate(x, ei_flat, ew):
    mesh = plsc.VectorSubcoreMesh(core_axis_name="c", subcore_axis_name="s")

    @functools.partial(
        pl.kernel,
        out_type=jax.ShapeDtypeStruct((NC, N_PAD, D), jnp.float32),
        mesh=mesh,
        scratch_types=[
            pltpu.VMEM((NBUF, C), jnp.int32),
            pltpu.VMEM((NBUF, C), jnp.float32),
            pltpu.VMEM((NBUF, C), jnp.int32),
            pltpu.VMEM((NBUF, C, D), jnp.float32),
            pltpu.VMEM_SHARED((N_PAD, D), jnp.float32),
            [pltpu.SemaphoreType.DMA] * NBUF,
            [pltpu.SemaphoreType.DMA] * NBUF,
            [pltpu.SemaphoreType.DMA] * NBUF,
            [pltpu.SemaphoreType.DMA] * NBUF,
            [pltpu.SemaphoreType.DMA] * NBUF,
        ],
    )
    def agg(x_hbm, ei_hbm, ew_hbm, out_hbm,
            idx_v, ew_v, dst_v, rows_v, acc,
            semG, semS, semI, semE, semD):
        cid = lax.axis_index("c")
        sid = lax.axis_index("s")
        wid = cid * NS + sid

        zero16 = jnp.zeros((L,), jnp.float32)

        def zstore(i, _):
            for k in range(D // L):
                rows_v[0, i, pl.ds(k * L, L)] = zero16
            return 0

        lax.fori_loop(0, C, zstore, 0)
        rbase = sid * ROWS_PT
        for j in range(ROWS_PT // C):
            pltpu.sync_copy(rows_v.at[0], acc.at[pl.ds(rbase + j * C, C)])
        plsc.subcore_barrier()

        def load_meta(g, buf):
            b = pl.multiple_of(wid * EPW + g * C, 8)
            pltpu.async_copy(ei_hbm.at[pl.ds(b, C)], idx_v.at[buf], semI[buf])
            pltpu.async_copy(ew_hbm.at[pl.ds(b, C)], ew_v.at[buf], semE[buf])

        def load_dst(g, buf):
            b = pl.multiple_of(N_EDGES + wid * EPW + g * C, 8)
            pltpu.async_copy(ei_hbm.at[pl.ds(b, C)], dst_v.at[buf], semD[buf])

        def wait_src(buf):
            pltpu.make_async_copy(
                ei_hbm.at[pl.ds(0, C)], idx_v.at[buf], semI[buf]).wait()

        def wait_ew(buf):
            pltpu.make_async_copy(
                ew_hbm.at[pl.ds(0, C)], ew_v.at[buf], semE[buf]).wait()

        def wait_dst(buf):
            pltpu.make_async_copy(
                ei_hbm.at[pl.ds(0, C)], dst_v.at[buf], semD[buf]).wait()

        def start_gather(buf):
            pltpu.async_copy(x_hbm.at[idx_v.at[buf]], rows_v.at[buf],
                             semG[buf])

        def wait_gather(buf):
            pltpu.make_async_copy(
                x_hbm.at[idx_v.at[buf]], rows_v.at[buf], semG[buf]).wait()

        def start_scatter(buf):
            pltpu.async_copy(rows_v.at[buf], acc.at[dst_v.at[buf]], semS[buf],
                             add=True)

        def wait_scatter(buf):
            pltpu.make_async_copy(
                rows_v.at[buf], acc.at[dst_v.at[buf]], semS[buf]).wait()

        def scale(buf):
            def sbody(gg, _):
                wvec = ew_v[buf, pl.ds(gg * L, L)]
                for j in range(L):
                    w = wvec[j]
                    e = gg * L + j
                    for k in range(D // L):
                        sl = pl.ds(k * L, L)
                        rows_v[buf, e, sl] = rows_v[buf, e, sl] * w
                return 0

            lax.fori_loop(0, C // L, sbody, 0)

        load_meta(0, 0)
        load_meta(1, 1)
        load_meta(2, 2)
        load_dst(0, 0)
        wait_src(0)
        start_gather(0)
        wait_src(1)
        start_gather(1)

        @pl.loop(0, NCHUNK + 3, step=NBUF)
        def _(g0):
            for bb in range(NBUF):
                g = g0 + bb
                b1 = (bb + 1) % NBUF
                b2 = (bb + 2) % NBUF
                b3 = (bb + 3) % NBUF

                @pl.when(g < NCHUNK)
                def _():
                    @pl.when(g + 3 < NCHUNK)
                    def _():
                        load_meta(g + 3, b3)

                    @pl.when(g + 2 < NCHUNK)
                    def _():
                        @pl.when(g >= 2)
                        def _():
                            wait_scatter(b2)

                        wait_src(b2)
                        start_gather(b2)

                    @pl.when(g + 1 < NCHUNK)
                    def _():
                        load_dst(g + 1, b1)

                    wait_gather(bb)
                    wait_ew(bb)
                    scale(bb)
                    wait_dst(bb)
                    start_scatter(bb)

        for buf in range(NBUF):
            wait_scatter(buf)
        plsc.subcore_barrier()

        pltpu.sync_copy(acc.at[pl.ds(rbase, ROWS_PT)],
                        out_hbm.at[cid, pl.ds(rbase, ROWS_PT)])

    return agg(x, ei_flat, ew)


def _tc_combine(parts, W, b):
    R = 1000

    def body(p_ref, w_ref, b_ref, o_ref):
        s = p_ref[0] + p_ref[1]
        o_ref[...] = (
            jnp.dot(s, w_ref[...], preferred_element_type=jnp.float32)
            + b_ref[...]
        )

    return pl.pallas_call(
        body,
        grid=(N_NODES // R,),
        in_specs=[
            pl.BlockSpec((NC, R, D), lambda i: (0, i, 0)),
            pl.BlockSpec((D, D), lambda i: (0, 0)),
            pl.BlockSpec((1, D), lambda i: (0, 0)),
        ],
        out_specs=pl.BlockSpec((R, D), lambda i: (i, 0)),
        out_shape=jax.ShapeDtypeStruct((N_NODES, D), jnp.float32),
    )(parts, W, b.reshape(1, D))


def kernel(x, edge_index, edge_weight, W, b):
    ei_flat = edge_index.astype(jnp.int32).reshape(2 * N_EDGES)
    parts = _sc_aggregate(x, ei_flat, edge_weight)
    return _tc_combine(parts, W, b)
